# baseline (speedup 1.0000x reference)
import jax
import jax.numpy as jnp
from jax import lax
from jax.experimental import pallas as pl
from jax.experimental.pallas import tpu as pltpu
from jax.experimental.pallas import tpu_sc as plsc

NC = 2
NS = 16
L = 16
NW = NC * NS
B = 16384
D = 64
FIELD0 = 1_000_000

TILE = 128
TTILES = 2 * FIELD0 // TILE
SLABW = 768
CAP = 2048

BPW2 = B // NW
CH2 = 128


def _extract_body(tT_hbm, xu_hbm, xi_hbm, r_hbm,
                  xcols_v, mcol_v, mslot_v, pcol_v, pslot_v,
                  slab_v, rows_v, sem0, sem1, sem2):
    wid = lax.axis_index("s") * NC + lax.axis_index("c")
    lo = ((wid * TTILES) // NW) * TILE
    hi = (((wid + 1) * TTILES) // NW) * TILE
    span = hi - lo
    nslab = (span + SLABW - 1) // SLABW
    iota = lax.broadcasted_iota(jnp.int32, (L,), 0)

    def scan_field(cnt0, offset, slot0):
        def scan(v, cnt):
            cols = xcols_v[pl.ds(v * L, L)] + offset
            mask = (cols >= lo) & (cols < hi)
            plsc.store_compressed(mcol_v.at[pl.ds(cnt, L)], cols - lo, mask=mask)
            plsc.store_compressed(mslot_v.at[pl.ds(cnt, L)], slot0 + v * L + iota,
                                  mask=mask)
            n = plsc.all_reduce_population_count(mask)
            return jnp.minimum(cnt + n[0], CAP)

        return lax.fori_loop(0, B // L, scan, cnt0, unroll=False)

    sems = [sem0, sem1]

    def slab_lo(s):
        return jnp.minimum(s * SLABW, span - SLABW)

    def fire(s, par):
        pltpu.async_copy(tT_hbm.at[pl.ds(0, D // 2), pl.ds(lo + slab_lo(s), SLABW)],
                         slab_v.at[par, pl.ds(0, D // 2)], sems[par])
        pltpu.async_copy(tT_hbm.at[pl.ds(D // 2, D // 2), pl.ds(lo + slab_lo(s), SLABW)],
                         slab_v.at[par, pl.ds(D // 2, D // 2)], sems[par])

    def wait(s, par):
        pltpu.make_async_copy(tT_hbm.at[:, pl.ds(lo + slab_lo(s), SLABW)],
                              slab_v.at[par], sems[par]).wait()

    fire(0, 0)

    @pl.when(nslab > 1)
    def _():
        fire(1, 1)

    pltpu.sync_copy(xu_hbm, xcols_v)
    cnt = scan_field(0, 0, 0)
    pltpu.sync_copy(xi_hbm, xcols_v)
    cnt = scan_field(cnt, FIELD0, B)

    mcol_v[pl.ds(cnt, L)] = jnp.full((L,), 2 * FIELD0 + SLABW, jnp.int32)
    mslot_v[pl.ds(cnt, L)] = jnp.zeros((L,), jnp.int32)
    ngrp = (cnt + L - 1) // L

    def do_slab(s, par):
        wait(s, par)
        slo = slab_lo(s)

        def pscan(g, m):
            cols = mcol_v[pl.ds(g * L, L)]
            slots = mslot_v[pl.ds(g * L, L)]
            mask = (cols >= slo) & (cols < slo + SLABW)
            plsc.store_compressed(pcol_v.at[pl.ds(m, L)], cols - slo, mask=mask)
            plsc.store_compressed(pslot_v.at[pl.ds(m, L)], slots, mask=mask)
            n = plsc.all_reduce_population_count(mask)
            return jnp.minimum(m + n[0], CAP)

        m = lax.fori_loop(0, ngrp, pscan, 0, unroll=False)
        first_c = pcol_v[pl.ds(0, L)]
        first_s = pslot_v[pl.ds(0, L)]

        @pl.when(m > 0)
        def _():
            pcol_v[pl.ds(m, L)] = jnp.full((L,), first_c[0], jnp.int32)
            pslot_v[pl.ds(m, L)] = jnp.full((L,), first_s[0], jnp.int32)

            def grp(g, _):
                ccl = pcol_v[pl.ds(g * L, L)]
                slots = pslot_v[pl.ds(g * L, L)]

                def dim_body(d, _):
                    v = plsc.load_gather(slab_v.at[par],
                                         [jnp.full((L,), d, jnp.int32), ccl])
                    plsc.store_scatter(rows_v, [iota, jnp.full((L,), d, jnp.int32)], v)
                    return 0

                lax.fori_loop(0, D, dim_body, 0, unroll=False)
                pltpu.async_copy(rows_v, r_hbm.at[slots], sem2).wait()
                return 0

            lax.fori_loop(0, (m + L - 1) // L, grp, 0, unroll=False)

    def pair_body(t, _):
        s0 = 2 * t
        do_slab(s0, 0)

        @pl.when(s0 + 2 < nslab)
        def _():
            fire(s0 + 2, 0)

        @pl.when(s0 + 1 < nslab)
        def _():
            do_slab(s0 + 1, 1)

            @pl.when(s0 + 3 < nslab)
            def _():
                fire(s0 + 3, 1)

        return 0

    lax.fori_loop(0, (nslab + 1) // 2, pair_body, 0, unroll=False)


def _gmf_body(r_hbm, wb_hbm, out_hbm, u_v, i_v, wb_v, tile_v, out_v,
              sem0, sem1):
    wid = lax.axis_index("s") * NC + lax.axis_index("c")
    base = wid * BPW2
    iota = lax.broadcasted_iota(jnp.int32, (L,), 0)

    pltpu.sync_copy(wb_hbm, wb_v)

    sems = [sem0, sem1]

    def fire(c, par):
        pltpu.async_copy(r_hbm.at[pl.ds(base + c * CH2, CH2)], u_v.at[par],
                         sems[par])
        pltpu.async_copy(r_hbm.at[pl.ds(B + base + c * CH2, CH2)],
                         i_v.at[par], sems[par])

    def wait(c, par):
        pltpu.make_async_copy(r_hbm.at[pl.ds(base + c * CH2, CH2)],
                              u_v.at[par], sems[par]).wait()
        pltpu.make_async_copy(r_hbm.at[pl.ds(B + base + c * CH2, CH2)],
                              i_v.at[par], sems[par]).wait()

    fire(0, 0)
    fire(1, 1)

    w0 = wb_v[pl.ds(0, L)]
    w1 = wb_v[pl.ds(L, L)]
    w2 = wb_v[pl.ds(2 * L, L)]
    w3 = wb_v[pl.ds(3 * L, L)]
    bias = wb_v[pl.ds(D, L)]

    NCH = BPW2 // CH2

    def compute_chunk(c, par):
        ub = u_v.at[par]
        ib = i_v.at[par]

        def group_body(g, _):
            def elem_body(j, _):
                e = g * L + j
                p = (ub[e, pl.ds(0, L)] * ib[e, pl.ds(0, L)] * w0
                     + ub[e, pl.ds(L, L)] * ib[e, pl.ds(L, L)] * w1
                     + ub[e, pl.ds(2 * L, L)] * ib[e, pl.ds(2 * L, L)] * w2
                     + ub[e, pl.ds(3 * L, L)] * ib[e, pl.ds(3 * L, L)] * w3)
                plsc.store_scatter(tile_v, [iota * L + j], p)
                return 0

            lax.fori_loop(0, L, elem_body, 0, unroll=False)

            def row_sum(r, acc):
                return acc + tile_v[pl.ds(r * L, L)]

            acc = lax.fori_loop(0, L, row_sum, bias, unroll=False)
            sig = 1.0 / (1.0 + jnp.exp(-acc))
            out_v[pl.ds(c * CH2 + g * L, L)] = sig
            return 0

        lax.fori_loop(0, CH2 // L, group_body, 0, unroll=False)

    def pair_body(t, _):
        c0 = 2 * t
        wait(c0, 0)
        compute_chunk(c0, 0)

        @pl.when(c0 + 2 < NCH)
        def _():
            fire(c0 + 2, 0)

        wait(c0 + 1, 1)
        compute_chunk(c0 + 1, 1)

        @pl.when(c0 + 3 < NCH)
        def _():
            fire(c0 + 3, 1)

        return 0

    lax.fori_loop(0, NCH // 2, pair_body, 0, unroll=False)

    pltpu.sync_copy(out_v, out_hbm.at[pl.ds(base, BPW2)])


_MESH = dict(core_axis_name="c", subcore_axis_name="s",
             num_cores=NC, num_subcores=NS)


@jax.jit
def _ncf(xu, xi, tT, wb):
    mesh = plsc.VectorSubcoreMesh(**_MESH)
    r = pl.kernel(
        _extract_body,
        out_type=jax.ShapeDtypeStruct((2 * B, 128), jnp.float32),
        mesh=mesh,
        compiler_params=pltpu.CompilerParams(
            needs_layout_passes=False, use_tc_tiling_on_sc=True
        ),
        scratch_types=[
            pltpu.VMEM((B,), jnp.int32),
            pltpu.VMEM((CAP + L,), jnp.int32),
            pltpu.VMEM((CAP + L,), jnp.int32),
            pltpu.VMEM((CAP + L,), jnp.int32),
            pltpu.VMEM((CAP + L,), jnp.int32),
            pltpu.VMEM((2, D, SLABW), jnp.float32),
            pltpu.VMEM((L, 128), jnp.float32),
            pltpu.SemaphoreType.DMA,
            pltpu.SemaphoreType.DMA,
            pltpu.SemaphoreType.DMA,
        ],
    )(tT, xu, xi)

    mesh2 = plsc.VectorSubcoreMesh(**_MESH)
    return pl.kernel(
        _gmf_body,
        out_type=jax.ShapeDtypeStruct((B,), jnp.float32),
        mesh=mesh2,
        compiler_params=pltpu.CompilerParams(
            needs_layout_passes=False, use_tc_tiling_on_sc=True
        ),
        scratch_types=[
            pltpu.VMEM((2, CH2, 128), jnp.float32),
            pltpu.VMEM((2, CH2, 128), jnp.float32),
            pltpu.VMEM((80,), jnp.float32),
            pltpu.VMEM((L * L,), jnp.float32),
            pltpu.VMEM((BPW2,), jnp.float32),
            pltpu.SemaphoreType.DMA,
            pltpu.SemaphoreType.DMA,
        ],
    )(r, wb)


def kernel(x, table, W_fc, b_fc):
    x2 = x.astype(jnp.int32)
    xu = x2[:, 0]
    xi = x2[:, 1]
    tT = table.T
    wb = jnp.concatenate(
        [W_fc.reshape(D), jnp.broadcast_to(b_fc.astype(jnp.float32), (16,))]
    )
    out = _ncf(xu, xi, tT, wb)
    return (out, x)


# speedup vs baseline: 1.0232x; 1.0232x over previous
import jax
import jax.numpy as jnp
from jax import lax
from jax.experimental import pallas as pl
from jax.experimental.pallas import tpu as pltpu
from jax.experimental.pallas import tpu_sc as plsc

NC = 2
NS = 16
L = 16
NW = NC * NS
B = 16384
D = 64
FIELD0 = 1_000_000

TILE = 128
TTILES = 2 * FIELD0 // TILE
SLABW = 640
CAP = 4096

BPW2 = B // NW
CH2 = 128


def _extract_body(tT_hbm, xu_hbm, xi_hbm, r_hbm,
                  xcols_v, mcol_v, mslot_v, pcol_v, pslot_v,
                  slab_v, rows_v, sem0, sem1, sem2):
    wid = lax.axis_index("s") * NC + lax.axis_index("c")
    lo = ((wid * TTILES) // NW) * TILE
    hi = (((wid + 1) * TTILES) // NW) * TILE
    span = hi - lo
    nslab = (span + SLABW - 1) // SLABW
    iota = lax.broadcasted_iota(jnp.int32, (L,), 0)

    def scan_field(cnt0, offset, slot0):
        def scan(v, cnt):
            cols = xcols_v[pl.ds(v * L, L)] + offset
            mask = (cols >= lo) & (cols < hi)
            plsc.store_compressed(mcol_v.at[pl.ds(cnt, L)], cols - lo, mask=mask)
            plsc.store_compressed(mslot_v.at[pl.ds(cnt, L)], slot0 + v * L + iota,
                                  mask=mask)
            n = plsc.all_reduce_population_count(mask)
            return jnp.minimum(cnt + n[0], CAP)

        return lax.fori_loop(0, B // L, scan, cnt0, unroll=False)

    sems = [sem0, sem1]

    def slab_lo(s):
        return jnp.minimum(s * SLABW, span - SLABW)

    def fire(s, par):
        pltpu.async_copy(tT_hbm.at[pl.ds(0, D // 2), pl.ds(lo + slab_lo(s), SLABW)],
                         slab_v.at[par, pl.ds(0, D // 2)], sems[par])
        pltpu.async_copy(tT_hbm.at[pl.ds(D // 2, D // 2), pl.ds(lo + slab_lo(s), SLABW)],
                         slab_v.at[par, pl.ds(D // 2, D // 2)], sems[par])

    def wait(s, par):
        pltpu.make_async_copy(tT_hbm.at[:, pl.ds(lo + slab_lo(s), SLABW)],
                              slab_v.at[par], sems[par]).wait()

    fire(0, 0)

    @pl.when(nslab > 1)
    def _():
        fire(1, 1)

    pltpu.sync_copy(xu_hbm, xcols_v)
    cnt = scan_field(0, 0, 0)
    pltpu.sync_copy(xi_hbm, xcols_v)
    cnt = scan_field(cnt, FIELD0, B)

    mcol_v[pl.ds(cnt, L)] = jnp.full((L,), 2 * FIELD0 + SLABW, jnp.int32)
    mslot_v[pl.ds(cnt, L)] = jnp.zeros((L,), jnp.int32)
    ngrp = (cnt + L - 1) // L

    def do_slab(s, par):
        wait(s, par)
        slo = slab_lo(s)

        def pscan(g, m):
            cols = mcol_v[pl.ds(g * L, L)]
            slots = mslot_v[pl.ds(g * L, L)]
            mask = (cols >= slo) & (cols < slo + SLABW)
            plsc.store_compressed(pcol_v.at[pl.ds(m, L)], cols - slo, mask=mask)
            plsc.store_compressed(pslot_v.at[pl.ds(m, L)], slots, mask=mask)
            n = plsc.all_reduce_population_count(mask)
            return jnp.minimum(m + n[0], CAP)

        m = lax.fori_loop(0, ngrp, pscan, 0, unroll=False)
        first_c = pcol_v[pl.ds(0, L)]
        first_s = pslot_v[pl.ds(0, L)]

        @pl.when(m > 0)
        def _():
            pcol_v[pl.ds(m, L)] = jnp.full((L,), first_c[0], jnp.int32)
            pslot_v[pl.ds(m, L)] = jnp.full((L,), first_s[0], jnp.int32)

            def grp(g, _):
                ccl = pcol_v[pl.ds(g * L, L)]
                slots = pslot_v[pl.ds(g * L, L)]

                def dim_body(d, _):
                    v = plsc.load_gather(slab_v.at[par],
                                         [jnp.full((L,), d, jnp.int32), ccl])
                    plsc.store_scatter(rows_v, [iota, jnp.full((L,), d, jnp.int32)], v)
                    return 0

                lax.fori_loop(0, D, dim_body, 0, unroll=False)
                pltpu.async_copy(rows_v, r_hbm.at[slots], sem2).wait()
                return 0

            lax.fori_loop(0, (m + L - 1) // L, grp, 0, unroll=False)

    def pair_body(t, _):
        s0 = 2 * t
        do_slab(s0, 0)

        @pl.when(s0 + 2 < nslab)
        def _():
            fire(s0 + 2, 0)

        @pl.when(s0 + 1 < nslab)
        def _():
            do_slab(s0 + 1, 1)

            @pl.when(s0 + 3 < nslab)
            def _():
                fire(s0 + 3, 1)

        return 0

    lax.fori_loop(0, (nslab + 1) // 2, pair_body, 0, unroll=False)


def _gmf_body(r_hbm, wb_hbm, out_hbm, u_v, i_v, wb_v, tile_v, out_v,
              sem0, sem1):
    wid = lax.axis_index("s") * NC + lax.axis_index("c")
    base = wid * BPW2
    iota = lax.broadcasted_iota(jnp.int32, (L,), 0)

    pltpu.sync_copy(wb_hbm, wb_v)

    sems = [sem0, sem1]

    def fire(c, par):
        pltpu.async_copy(r_hbm.at[pl.ds(base + c * CH2, CH2)], u_v.at[par],
                         sems[par])
        pltpu.async_copy(r_hbm.at[pl.ds(B + base + c * CH2, CH2)],
                         i_v.at[par], sems[par])

    def wait(c, par):
        pltpu.make_async_copy(r_hbm.at[pl.ds(base + c * CH2, CH2)],
                              u_v.at[par], sems[par]).wait()
        pltpu.make_async_copy(r_hbm.at[pl.ds(B + base + c * CH2, CH2)],
                              i_v.at[par], sems[par]).wait()

    fire(0, 0)
    fire(1, 1)

    w0 = wb_v[pl.ds(0, L)]
    w1 = wb_v[pl.ds(L, L)]
    w2 = wb_v[pl.ds(2 * L, L)]
    w3 = wb_v[pl.ds(3 * L, L)]
    bias = wb_v[pl.ds(D, L)]

    NCH = BPW2 // CH2

    def compute_chunk(c, par):
        ub = u_v.at[par]
        ib = i_v.at[par]

        def group_body(g, _):
            def elem_body(j, _):
                e = g * L + j
                p = (ub[e, pl.ds(0, L)] * ib[e, pl.ds(0, L)] * w0
                     + ub[e, pl.ds(L, L)] * ib[e, pl.ds(L, L)] * w1
                     + ub[e, pl.ds(2 * L, L)] * ib[e, pl.ds(2 * L, L)] * w2
                     + ub[e, pl.ds(3 * L, L)] * ib[e, pl.ds(3 * L, L)] * w3)
                plsc.store_scatter(tile_v, [iota * L + j], p)
                return 0

            lax.fori_loop(0, L, elem_body, 0, unroll=False)

            def row_sum(r, acc):
                return acc + tile_v[pl.ds(r * L, L)]

            acc = lax.fori_loop(0, L, row_sum, bias, unroll=False)
            sig = 1.0 / (1.0 + jnp.exp(-acc))
            out_v[pl.ds(c * CH2 + g * L, L)] = sig
            return 0

        lax.fori_loop(0, CH2 // L, group_body, 0, unroll=False)

    def pair_body(t, _):
        c0 = 2 * t
        wait(c0, 0)
        compute_chunk(c0, 0)

        @pl.when(c0 + 2 < NCH)
        def _():
            fire(c0 + 2, 0)

        wait(c0 + 1, 1)
        compute_chunk(c0 + 1, 1)

        @pl.when(c0 + 3 < NCH)
        def _():
            fire(c0 + 3, 1)

        return 0

    lax.fori_loop(0, NCH // 2, pair_body, 0, unroll=False)

    pltpu.sync_copy(out_v, out_hbm.at[pl.ds(base, BPW2)])


_MESH = dict(core_axis_name="c", subcore_axis_name="s",
             num_cores=NC, num_subcores=NS)


@jax.jit
def _ncf(xu, xi, tT, wb):
    mesh = plsc.VectorSubcoreMesh(**_MESH)
    r = pl.kernel(
        _extract_body,
        out_type=jax.ShapeDtypeStruct((2 * B, 128), jnp.float32),
        mesh=mesh,
        compiler_params=pltpu.CompilerParams(
            needs_layout_passes=False, use_tc_tiling_on_sc=True
        ),
        scratch_types=[
            pltpu.VMEM((B,), jnp.int32),
            pltpu.VMEM((CAP + L,), jnp.int32),
            pltpu.VMEM((CAP + L,), jnp.int32),
            pltpu.VMEM((CAP + L,), jnp.int32),
            pltpu.VMEM((CAP + L,), jnp.int32),
            pltpu.VMEM((2, D, SLABW), jnp.float32),
            pltpu.VMEM((L, 128), jnp.float32),
            pltpu.SemaphoreType.DMA,
            pltpu.SemaphoreType.DMA,
            pltpu.SemaphoreType.DMA,
        ],
    )(tT, xu, xi)

    mesh2 = plsc.VectorSubcoreMesh(**_MESH)
    return pl.kernel(
        _gmf_body,
        out_type=jax.ShapeDtypeStruct((B,), jnp.float32),
        mesh=mesh2,
        compiler_params=pltpu.CompilerParams(
            needs_layout_passes=False, use_tc_tiling_on_sc=True
        ),
        scratch_types=[
            pltpu.VMEM((2, CH2, 128), jnp.float32),
            pltpu.VMEM((2, CH2, 128), jnp.float32),
            pltpu.VMEM((80,), jnp.float32),
            pltpu.VMEM((L * L,), jnp.float32),
            pltpu.VMEM((BPW2,), jnp.float32),
            pltpu.SemaphoreType.DMA,
            pltpu.SemaphoreType.DMA,
        ],
    )(r, wb)


def kernel(x, table, W_fc, b_fc):
    x2 = x.astype(jnp.int32)
    xu = x2[:, 0]
    xi = x2[:, 1]
    tT = table.T
    wb = jnp.concatenate(
        [W_fc.reshape(D), jnp.broadcast_to(b_fc.astype(jnp.float32), (16,))]
    )
    out = _ncf(xu, xi, tT, wb)
    return (out, x)


# speedup vs baseline: 1.0892x; 1.0645x over previous
import jax
import jax.numpy as jnp
from jax import lax
from jax.experimental import pallas as pl
from jax.experimental.pallas import tpu as pltpu
from jax.experimental.pallas import tpu_sc as plsc

NC = 2
NS = 16
L = 16
NW = NC * NS
B = 16384
D = 64
FIELD0 = 1_000_000

TILE = 128
TTILES = 2 * FIELD0 // TILE
SLABW = 512
CAP = 2048

BPW2 = B // NW
CH2 = 128


def _extract_body(tT_hbm, xu_hbm, xi_hbm, r_hbm,
                  xcols_v, mcol_v, mslot_v, pcol_v, pslot_v,
                  slab_v, rows_v, sem0, sem1, sem2x, sem2):
    wid = lax.axis_index("s") * NC + lax.axis_index("c")
    lo = ((wid * TTILES) // NW) * TILE
    hi = (((wid + 1) * TTILES) // NW) * TILE
    span = hi - lo
    nslab = (span + SLABW - 1) // SLABW
    iota = lax.broadcasted_iota(jnp.int32, (L,), 0)

    def scan_field(cnt0, offset, slot0):
        def scan(v, cnt):
            cols = xcols_v[pl.ds(v * L, L)] + offset
            mask = (cols >= lo) & (cols < hi)
            plsc.store_compressed(mcol_v.at[pl.ds(cnt, L)], cols - lo, mask=mask)
            plsc.store_compressed(mslot_v.at[pl.ds(cnt, L)], slot0 + v * L + iota,
                                  mask=mask)
            n = plsc.all_reduce_population_count(mask)
            return jnp.minimum(cnt + n[0], CAP)

        return lax.fori_loop(0, B // L, scan, cnt0, unroll=False)

    sems = [sem0, sem1, sem2x]

    def slab_lo(s):
        return jnp.minimum(s * SLABW, span - SLABW)

    def fire(s, par):
        pltpu.async_copy(tT_hbm.at[pl.ds(0, D // 2), pl.ds(lo + slab_lo(s), SLABW)],
                         slab_v.at[par, pl.ds(0, D // 2)], sems[par])
        pltpu.async_copy(tT_hbm.at[pl.ds(D // 2, D // 2), pl.ds(lo + slab_lo(s), SLABW)],
                         slab_v.at[par, pl.ds(D // 2, D // 2)], sems[par])

    def wait(s, par):
        pltpu.make_async_copy(tT_hbm.at[:, pl.ds(lo + slab_lo(s), SLABW)],
                              slab_v.at[par], sems[par]).wait()

    fire(0, 0)

    @pl.when(nslab > 1)
    def _():
        fire(1, 1)

    @pl.when(nslab > 2)
    def _():
        fire(2, 2)

    pltpu.sync_copy(xu_hbm, xcols_v)
    cnt = scan_field(0, 0, 0)
    pltpu.sync_copy(xi_hbm, xcols_v)
    cnt = scan_field(cnt, FIELD0, B)

    mcol_v[pl.ds(cnt, L)] = jnp.full((L,), 2 * FIELD0 + SLABW, jnp.int32)
    mslot_v[pl.ds(cnt, L)] = jnp.zeros((L,), jnp.int32)
    ngrp = (cnt + L - 1) // L

    def do_slab(s, par):
        wait(s, par)
        slo = slab_lo(s)

        def pscan(g, m):
            cols = mcol_v[pl.ds(g * L, L)]
            slots = mslot_v[pl.ds(g * L, L)]
            mask = (cols >= slo) & (cols < slo + SLABW)
            plsc.store_compressed(pcol_v.at[pl.ds(m, L)], cols - slo, mask=mask)
            plsc.store_compressed(pslot_v.at[pl.ds(m, L)], slots, mask=mask)
            n = plsc.all_reduce_population_count(mask)
            return jnp.minimum(m + n[0], CAP)

        m = lax.fori_loop(0, ngrp, pscan, 0, unroll=False)
        first_c = pcol_v[pl.ds(0, L)]
        first_s = pslot_v[pl.ds(0, L)]

        @pl.when(m > 0)
        def _():
            pcol_v[pl.ds(m, L)] = jnp.full((L,), first_c[0], jnp.int32)
            pslot_v[pl.ds(m, L)] = jnp.full((L,), first_s[0], jnp.int32)

            def grp(g, _):
                ccl = pcol_v[pl.ds(g * L, L)]
                slots = pslot_v[pl.ds(g * L, L)]

                def dim_body(d, _):
                    v = plsc.load_gather(slab_v.at[par],
                                         [jnp.full((L,), d, jnp.int32), ccl])
                    plsc.store_scatter(rows_v, [iota, jnp.full((L,), d, jnp.int32)], v)
                    return 0

                lax.fori_loop(0, D, dim_body, 0, unroll=False)
                pltpu.async_copy(rows_v, r_hbm.at[slots], sem2).wait()
                return 0

            lax.fori_loop(0, (m + L - 1) // L, grp, 0, unroll=False)

    def triple_body(t, _):
        s0 = 3 * t
        for k in range(3):
            @pl.when(s0 + k < nslab)
            def _(k=k):
                do_slab(s0 + k, k)

                @pl.when(s0 + k + 3 < nslab)
                def _():
                    fire(s0 + k + 3, k)

        return 0

    lax.fori_loop(0, (nslab + 2) // 3, triple_body, 0, unroll=False)


def _gmf_body(r_hbm, wb_hbm, out_hbm, u_v, i_v, wb_v, tile_v, out_v,
              sem0, sem1):
    wid = lax.axis_index("s") * NC + lax.axis_index("c")
    base = wid * BPW2
    iota = lax.broadcasted_iota(jnp.int32, (L,), 0)

    pltpu.sync_copy(wb_hbm, wb_v)

    sems = [sem0, sem1]

    def fire(c, par):
        pltpu.async_copy(r_hbm.at[pl.ds(base + c * CH2, CH2)], u_v.at[par],
                         sems[par])
        pltpu.async_copy(r_hbm.at[pl.ds(B + base + c * CH2, CH2)],
                         i_v.at[par], sems[par])

    def wait(c, par):
        pltpu.make_async_copy(r_hbm.at[pl.ds(base + c * CH2, CH2)],
                              u_v.at[par], sems[par]).wait()
        pltpu.make_async_copy(r_hbm.at[pl.ds(B + base + c * CH2, CH2)],
                              i_v.at[par], sems[par]).wait()

    fire(0, 0)
    fire(1, 1)

    w0 = wb_v[pl.ds(0, L)]
    w1 = wb_v[pl.ds(L, L)]
    w2 = wb_v[pl.ds(2 * L, L)]
    w3 = wb_v[pl.ds(3 * L, L)]
    bias = wb_v[pl.ds(D, L)]

    NCH = BPW2 // CH2

    def compute_chunk(c, par):
        ub = u_v.at[par]
        ib = i_v.at[par]

        def group_body(g, _):
            def elem_body(j, _):
                e = g * L + j
                p = (ub[e, pl.ds(0, L)] * ib[e, pl.ds(0, L)] * w0
                     + ub[e, pl.ds(L, L)] * ib[e, pl.ds(L, L)] * w1
                     + ub[e, pl.ds(2 * L, L)] * ib[e, pl.ds(2 * L, L)] * w2
                     + ub[e, pl.ds(3 * L, L)] * ib[e, pl.ds(3 * L, L)] * w3)
                plsc.store_scatter(tile_v, [iota * L + j], p)
                return 0

            lax.fori_loop(0, L, elem_body, 0, unroll=False)

            def row_sum(r, acc):
                return acc + tile_v[pl.ds(r * L, L)]

            acc = lax.fori_loop(0, L, row_sum, bias, unroll=False)
            sig = 1.0 / (1.0 + jnp.exp(-acc))
            out_v[pl.ds(c * CH2 + g * L, L)] = sig
            return 0

        lax.fori_loop(0, CH2 // L, group_body, 0, unroll=False)

    def pair_body(t, _):
        c0 = 2 * t
        wait(c0, 0)
        compute_chunk(c0, 0)

        @pl.when(c0 + 2 < NCH)
        def _():
            fire(c0 + 2, 0)

        wait(c0 + 1, 1)
        compute_chunk(c0 + 1, 1)

        @pl.when(c0 + 3 < NCH)
        def _():
            fire(c0 + 3, 1)

        return 0

    lax.fori_loop(0, NCH // 2, pair_body, 0, unroll=False)

    pltpu.sync_copy(out_v, out_hbm.at[pl.ds(base, BPW2)])


_MESH = dict(core_axis_name="c", subcore_axis_name="s",
             num_cores=NC, num_subcores=NS)


@jax.jit
def _ncf(xu, xi, tT, wb):
    mesh = plsc.VectorSubcoreMesh(**_MESH)
    r = pl.kernel(
        _extract_body,
        out_type=jax.ShapeDtypeStruct((2 * B, 128), jnp.float32),
        mesh=mesh,
        compiler_params=pltpu.CompilerParams(
            needs_layout_passes=False, use_tc_tiling_on_sc=True
        ),
        scratch_types=[
            pltpu.VMEM((B,), jnp.int32),
            pltpu.VMEM((CAP + L,), jnp.int32),
            pltpu.VMEM((CAP + L,), jnp.int32),
            pltpu.VMEM((CAP + L,), jnp.int32),
            pltpu.VMEM((CAP + L,), jnp.int32),
            pltpu.VMEM((3, D, SLABW), jnp.float32),
            pltpu.VMEM((L, 128), jnp.float32),
            pltpu.SemaphoreType.DMA,
            pltpu.SemaphoreType.DMA,
            pltpu.SemaphoreType.DMA,
            pltpu.SemaphoreType.DMA,
        ],
    )(tT, xu, xi)

    mesh2 = plsc.VectorSubcoreMesh(**_MESH)
    return pl.kernel(
        _gmf_body,
        out_type=jax.ShapeDtypeStruct((B,), jnp.float32),
        mesh=mesh2,
        compiler_params=pltpu.CompilerParams(
            needs_layout_passes=False, use_tc_tiling_on_sc=True
        ),
        scratch_types=[
            pltpu.VMEM((2, CH2, 128), jnp.float32),
            pltpu.VMEM((2, CH2, 128), jnp.float32),
            pltpu.VMEM((80,), jnp.float32),
            pltpu.VMEM((L * L,), jnp.float32),
            pltpu.VMEM((BPW2,), jnp.float32),
            pltpu.SemaphoreType.DMA,
            pltpu.SemaphoreType.DMA,
        ],
    )(r, wb)


def kernel(x, table, W_fc, b_fc):
    x2 = x.astype(jnp.int32)
    xu = x2[:, 0]
    xi = x2[:, 1]
    tT = table.T
    wb = jnp.concatenate(
        [W_fc.reshape(D), jnp.broadcast_to(b_fc.astype(jnp.float32), (16,))]
    )
    out = _ncf(xu, xi, tT, wb)
    return (out, x)
